# baseline (speedup 1.0000x reference)
import functools

import jax
import jax.numpy as jnp
from jax import lax
from jax.experimental import pallas as pl
from jax.experimental.pallas import tpu as pltpu
from jax.experimental.pallas import tpu_sc as plsc

E = 8
T = 2048
H = 1024
I = 768
G = 128
NT = (2 * T + E * (G - 1) + G - 1) // G
C = NT * G
TB = 512
NW = 32
TPW = T // NW
RCH = 32


def _router_body(x_ref, gate_ref, dests_ref, wbr_ref, counts_ref, carry_ref,
                 ls_ref):
    i = pl.program_id(0)

    @pl.when(i == 0)
    def _():
        carry_ref[...] = jnp.zeros_like(carry_ref)

    x = x_ref[...]
    logits = lax.dot_general(x, gate_ref[...], (((1,), (1,)), ((), ())),
                             preferred_element_type=jnp.float32)
    eio = lax.broadcasted_iota(jnp.int32, logits.shape, 1)
    m1 = jnp.max(logits, axis=1, keepdims=True)
    i1 = jnp.min(jnp.where(logits == m1, eio, E), axis=1, keepdims=True)
    l2 = jnp.where(eio == i1, -jnp.inf, logits)
    m2 = jnp.max(l2, axis=1, keepdims=True)
    i2 = jnp.min(jnp.where(l2 == m2, eio, E), axis=1, keepdims=True)
    w1 = 1.0 / (1.0 + jnp.exp(m2 - m1))
    w2 = 1.0 - w1

    o1 = (eio == i1).astype(jnp.float32)
    o2 = (eio == i2).astype(jnp.float32)
    osum = o1 + o2
    @pl.when(i == 0)
    def _():
        rr = lax.broadcasted_iota(jnp.int32, (TB, TB), 0)
        cc = lax.broadcasted_iota(jnp.int32, (TB, TB), 1)
        ls_ref[...] = (rr > cc).astype(jnp.float32)

    ls = ls_ref[...]
    carry = carry_ref[...]
    cum = lax.dot_general(ls, osum, (((1,), (0,)), ((), ())),
                          preferred_element_type=jnp.float32) + carry
    cb1 = jnp.sum(o1 * cum, axis=1, keepdims=True)
    cb2 = jnp.sum(o2 * cum, axis=1, keepdims=True)
    carry_ref[...] = carry + jnp.sum(osum, axis=0, keepdims=True)

    meta = jnp.concatenate(
        [cb1, cb2, i1.astype(jnp.float32), i2.astype(jnp.float32), w1, w2,
         jnp.zeros((TB, 2), jnp.float32)], axis=1)
    dests_ref[pl.ds(i * TB, TB), :] = meta

    @pl.when(i == T // TB - 1)
    def _():
        cnt = carry_ref[...]
        counts_ref[...] = cnt
        padded = jnp.floor((cnt + (G - 1)) / G) * G
        rr8 = lax.broadcasted_iota(jnp.int32, (E, E), 0)
        cc8 = lax.broadcasted_iota(jnp.int32, (E, E), 1)
        uincl = (rr8 <= cc8).astype(jnp.float32)
        bounds = lax.dot_general(padded, uincl, (((1,), (0,)), ((), ())),
                                 preferred_element_type=jnp.float32)
        offs = bounds - padded
        allm = dests_ref[...]
        eiof = lax.broadcasted_iota(jnp.int32, (T, E), 1).astype(jnp.float32)
        oo1 = (eiof == allm[:, 2:3]).astype(jnp.float32)
        oo2 = (eiof == allm[:, 3:4]).astype(jnp.float32)
        d1 = allm[:, 0:1] + jnp.sum(oo1 * offs, axis=1, keepdims=True)
        d2 = allm[:, 1:2] + jnp.sum(oo2 * offs, axis=1, keepdims=True)
        dests_ref[...] = jnp.concatenate(
            [d1, d2, allm[:, 2:]], axis=1)
        wbr_ref[...] = jnp.concatenate(
            [jnp.broadcast_to(allm[:, 4:5], (T, 16)),
             jnp.broadcast_to(allm[:, 5:6], (T, 16))], axis=1)


def _router(x, gate_w):
    return pl.pallas_call(
        _router_body,
        grid=(T // TB,),
        in_specs=[
            pl.BlockSpec((TB, H), lambda i: (i, 0)),
            pl.BlockSpec((E, H), lambda i: (0, 0)),
        ],
        out_specs=[
            pl.BlockSpec((T, 8), lambda i: (0, 0)),
            pl.BlockSpec((T, 32), lambda i: (0, 0)),
            pl.BlockSpec((1, E), lambda i: (0, 0)),
        ],
        out_shape=[
            jax.ShapeDtypeStruct((T, 8), jnp.float32),
            jax.ShapeDtypeStruct((T, 32), jnp.float32),
            jax.ShapeDtypeStruct((1, E), jnp.float32),
        ],
        scratch_shapes=[pltpu.VMEM((1, E), jnp.float32),
                        pltpu.VMEM((TB, TB), jnp.float32)],
        compiler_params=pltpu.CompilerParams(
            dimension_semantics=("arbitrary",)),
    )(x, gate_w)


def _dispatch_body(x_hbm, d1_hbm, d2_hbm, xs_hbm, da_v, db_v, rows_v, sem,
                   semb):
    wid = lax.axis_index("s") * 2 + lax.axis_index("c")
    base = wid * TPW
    for h in range(TPW // RCH):
        sub = base + h * RCH
        pltpu.sync_copy(x_hbm.at[pl.ds(sub, RCH)], rows_v)
        pltpu.sync_copy(d1_hbm.at[pl.ds(sub, RCH)], da_v)
        pltpu.sync_copy(d2_hbm.at[pl.ds(sub, RCH)], db_v)
        cpa = pltpu.async_copy(rows_v, xs_hbm.at[da_v], sem)
        cpb = pltpu.async_copy(rows_v, xs_hbm.at[db_v], semb)
        cpa.wait()
        cpb.wait()


def _dispatch(x, d1, d2):
    run = pl.kernel(
        _dispatch_body,
        out_type=jax.ShapeDtypeStruct((C, H), jnp.float32),
        mesh=plsc.VectorSubcoreMesh(core_axis_name="c", subcore_axis_name="s",
                                    num_cores=2, num_subcores=16),
        scratch_types=[
            pltpu.VMEM((RCH,), jnp.int32),
            pltpu.VMEM((RCH,), jnp.int32),
            pltpu.VMEM((RCH, H), jnp.float32),
            pltpu.SemaphoreType.DMA,
            pltpu.SemaphoreType.DMA,
        ],
    )
    return run(x, d1, d2)


def _glu_body(tid_ref, valid_ref, xs_ref, gw_ref, uw_ref, dw_ref, ys_ref):
    j = pl.program_id(0)

    @pl.when(valid_ref[j] > 0)
    def _():
        xb = xs_ref[...].astype(jnp.bfloat16)
        g = lax.dot_general(xb, gw_ref[0].astype(jnp.bfloat16),
                            (((1,), (1,)), ((), ())),
                            preferred_element_type=jnp.float32)
        u = lax.dot_general(xb, uw_ref[0].astype(jnp.bfloat16),
                            (((1,), (1,)), ((), ())),
                            preferred_element_type=jnp.float32)
        hh = ((g * jax.nn.sigmoid(g)) * u).astype(jnp.bfloat16)
        ys_ref[...] = lax.dot_general(hh, dw_ref[0].astype(jnp.bfloat16),
                                      (((1,), (1,)), ((), ())),
                                      preferred_element_type=jnp.float32)


def _grouped_glu(tile_eid, tile_valid, xs, gate_proj_w, up_proj_w,
                 down_proj_w):
    grid_spec = pltpu.PrefetchScalarGridSpec(
        num_scalar_prefetch=2,
        grid=(NT,),
        in_specs=[
            pl.BlockSpec((G, H), lambda j, tid, val: (j, 0)),
            pl.BlockSpec((1, I, H), lambda j, tid, val: (tid[j], 0, 0)),
            pl.BlockSpec((1, I, H), lambda j, tid, val: (tid[j], 0, 0)),
            pl.BlockSpec((1, H, I), lambda j, tid, val: (tid[j], 0, 0)),
        ],
        out_specs=pl.BlockSpec((G, H), lambda j, tid, val: (j, 0)),
    )
    return pl.pallas_call(
        _glu_body,
        grid_spec=grid_spec,
        out_shape=jax.ShapeDtypeStruct((C, H), jnp.float32),
        compiler_params=pltpu.CompilerParams(
            dimension_semantics=("arbitrary",)),
    )(tile_eid, tile_valid, xs, gate_proj_w, up_proj_w, down_proj_w)


def _combine_body(ys_hbm, d1_hbm, d2_hbm, wbr_hbm,
                  out_hbm, da_v, db_v, wb_v, rowsa_v, rowsb_v,
                  out_v, sem, semb):
    wid = lax.axis_index("s") * 2 + lax.axis_index("c")
    base = wid * TPW
    for h in range(TPW // RCH):
        sub = base + h * RCH
        pltpu.sync_copy(d1_hbm.at[pl.ds(sub, RCH)], da_v)
        pltpu.sync_copy(d2_hbm.at[pl.ds(sub, RCH)], db_v)
        pltpu.sync_copy(wbr_hbm.at[pl.ds(sub, RCH)], wb_v)
        cpa = pltpu.async_copy(ys_hbm.at[da_v], rowsa_v, sem)
        cpb = pltpu.async_copy(ys_hbm.at[db_v], rowsb_v, semb)
        cpa.wait()
        cpb.wait()

        def jbody(j, _):
            wa = wb_v[j, pl.ds(0, 16)]
            wb = wb_v[j, pl.ds(16, 16)]
            for c in range(H // 16):
                sl = pl.ds(c * 16, 16)
                out_v[j, sl] = rowsa_v[j, sl] * wa + rowsb_v[j, sl] * wb
            return 0

        lax.fori_loop(0, RCH, jbody, 0)
        pltpu.sync_copy(out_v, out_hbm.at[pl.ds(sub, RCH)])


def _combine(ys, d1, d2, wbr):
    run = pl.kernel(
        _combine_body,
        out_type=jax.ShapeDtypeStruct((T, H), jnp.float32),
        mesh=plsc.VectorSubcoreMesh(core_axis_name="c", subcore_axis_name="s",
                                    num_cores=2, num_subcores=16),
        scratch_types=[
            pltpu.VMEM((RCH,), jnp.int32),
            pltpu.VMEM((RCH,), jnp.int32),
            pltpu.VMEM((RCH, 32), jnp.float32),
            pltpu.VMEM((RCH, H), jnp.float32),
            pltpu.VMEM((RCH, H), jnp.float32),
            pltpu.VMEM((RCH, H), jnp.float32),
            pltpu.SemaphoreType.DMA,
            pltpu.SemaphoreType.DMA,
        ],
    )
    return run(ys, d1, d2, wbr)


def _round_i32(x):
    return (x + 0.5).astype(jnp.int32)


def kernel(hidden_states, gate_w, gate_proj_w, up_proj_w, down_proj_w):
    b, s, hd = hidden_states.shape
    x = hidden_states.reshape(-1, hd)

    dests, wbr, counts_w = _router(x, gate_w)
    d1 = _round_i32(dests[:, 0])
    d2 = _round_i32(dests[:, 1])
    counts = _round_i32(counts_w[0])

    padded = ((counts + G - 1) // G) * G
    bounds = jnp.cumsum(padded)
    tiles_active = bounds[-1] // G
    tj = jnp.arange(NT, dtype=jnp.int32)
    eid_raw = jnp.sum((tj[:, None] * G >= bounds[None, :]).astype(jnp.int32),
                      axis=1)
    tile_valid = (tj < tiles_active).astype(jnp.int32)
    last_e = eid_raw[jnp.maximum(tiles_active - 1, 0)]
    tile_eid = jnp.where(tile_valid > 0, eid_raw, last_e)

    xs = _dispatch(x, d1, d2)
    ys = _grouped_glu(tile_eid, tile_valid, xs, gate_proj_w, up_proj_w,
                      down_proj_w)
    out = _combine(ys, d1, d2, wbr)
    return out.reshape(b, s, hd)


# speedup vs baseline: 1.2069x; 1.2069x over previous
import functools

import jax
import jax.numpy as jnp
from jax import lax
from jax.experimental import pallas as pl
from jax.experimental.pallas import tpu as pltpu
from jax.experimental.pallas import tpu_sc as plsc

E = 8
T = 2048
H = 1024
I = 768
G = 256
NT = (2 * T + E * (G - 1) + G - 1) // G
C = NT * G
TB = 512
NW = 32
TPW = T // NW
RCH = 32


def _router_body(x_ref, gate_ref, dests_ref, wbr_ref, counts_ref, carry_ref,
                 ls_ref):
    i = pl.program_id(0)

    @pl.when(i == 0)
    def _():
        carry_ref[...] = jnp.zeros_like(carry_ref)

    x = x_ref[...]
    logits = lax.dot_general(x, gate_ref[...], (((1,), (1,)), ((), ())),
                             preferred_element_type=jnp.float32)
    eio = lax.broadcasted_iota(jnp.int32, logits.shape, 1)
    m1 = jnp.max(logits, axis=1, keepdims=True)
    i1 = jnp.min(jnp.where(logits == m1, eio, E), axis=1, keepdims=True)
    l2 = jnp.where(eio == i1, -jnp.inf, logits)
    m2 = jnp.max(l2, axis=1, keepdims=True)
    i2 = jnp.min(jnp.where(l2 == m2, eio, E), axis=1, keepdims=True)
    w1 = 1.0 / (1.0 + jnp.exp(m2 - m1))
    w2 = 1.0 - w1

    o1 = (eio == i1).astype(jnp.float32)
    o2 = (eio == i2).astype(jnp.float32)
    osum = o1 + o2
    @pl.when(i == 0)
    def _():
        rr = lax.broadcasted_iota(jnp.int32, (TB, TB), 0)
        cc = lax.broadcasted_iota(jnp.int32, (TB, TB), 1)
        ls_ref[...] = (rr > cc).astype(jnp.float32)

    ls = ls_ref[...]
    carry = carry_ref[...]
    cum = lax.dot_general(ls, osum, (((1,), (0,)), ((), ())),
                          preferred_element_type=jnp.float32) + carry
    cb1 = jnp.sum(o1 * cum, axis=1, keepdims=True)
    cb2 = jnp.sum(o2 * cum, axis=1, keepdims=True)
    carry_ref[...] = carry + jnp.sum(osum, axis=0, keepdims=True)

    meta = jnp.concatenate(
        [cb1, cb2, i1.astype(jnp.float32), i2.astype(jnp.float32), w1, w2,
         jnp.zeros((TB, 2), jnp.float32)], axis=1)
    dests_ref[pl.ds(i * TB, TB), :] = meta

    @pl.when(i == T // TB - 1)
    def _():
        cnt = carry_ref[...]
        counts_ref[...] = cnt
        padded = jnp.floor((cnt + (G - 1)) / G) * G
        rr8 = lax.broadcasted_iota(jnp.int32, (E, E), 0)
        cc8 = lax.broadcasted_iota(jnp.int32, (E, E), 1)
        uincl = (rr8 <= cc8).astype(jnp.float32)
        bounds = lax.dot_general(padded, uincl, (((1,), (0,)), ((), ())),
                                 preferred_element_type=jnp.float32)
        offs = bounds - padded
        allm = dests_ref[...]
        eiof = lax.broadcasted_iota(jnp.int32, (T, E), 1).astype(jnp.float32)
        oo1 = (eiof == allm[:, 2:3]).astype(jnp.float32)
        oo2 = (eiof == allm[:, 3:4]).astype(jnp.float32)
        d1 = allm[:, 0:1] + jnp.sum(oo1 * offs, axis=1, keepdims=True)
        d2 = allm[:, 1:2] + jnp.sum(oo2 * offs, axis=1, keepdims=True)
        dests_ref[...] = jnp.concatenate(
            [d1, d2, allm[:, 2:]], axis=1)
        wbr_ref[...] = jnp.concatenate(
            [jnp.broadcast_to(allm[:, 4:5], (T, 16)),
             jnp.broadcast_to(allm[:, 5:6], (T, 16))], axis=1)


def _router(x, gate_w):
    return pl.pallas_call(
        _router_body,
        grid=(T // TB,),
        in_specs=[
            pl.BlockSpec((TB, H), lambda i: (i, 0)),
            pl.BlockSpec((E, H), lambda i: (0, 0)),
        ],
        out_specs=[
            pl.BlockSpec((T, 8), lambda i: (0, 0)),
            pl.BlockSpec((T, 32), lambda i: (0, 0)),
            pl.BlockSpec((1, E), lambda i: (0, 0)),
        ],
        out_shape=[
            jax.ShapeDtypeStruct((T, 8), jnp.float32),
            jax.ShapeDtypeStruct((T, 32), jnp.float32),
            jax.ShapeDtypeStruct((1, E), jnp.float32),
        ],
        scratch_shapes=[pltpu.VMEM((1, E), jnp.float32),
                        pltpu.VMEM((TB, TB), jnp.float32)],
        compiler_params=pltpu.CompilerParams(
            dimension_semantics=("arbitrary",)),
    )(x, gate_w)


def _dispatch_body(x_hbm, d1_hbm, d2_hbm, xs_hbm, da_v, db_v, rows_v, sem,
                   semb):
    wid = lax.axis_index("s") * 2 + lax.axis_index("c")
    base = wid * TPW
    for h in range(TPW // RCH):
        sub = base + h * RCH
        pltpu.sync_copy(x_hbm.at[pl.ds(sub, RCH)], rows_v)
        pltpu.sync_copy(d1_hbm.at[pl.ds(sub, RCH)], da_v)
        pltpu.sync_copy(d2_hbm.at[pl.ds(sub, RCH)], db_v)
        cpa = pltpu.async_copy(rows_v, xs_hbm.at[da_v], sem)
        cpb = pltpu.async_copy(rows_v, xs_hbm.at[db_v], semb)
        cpa.wait()
        cpb.wait()


def _dispatch(x, d1, d2):
    run = pl.kernel(
        _dispatch_body,
        out_type=jax.ShapeDtypeStruct((C, H), jnp.float32),
        mesh=plsc.VectorSubcoreMesh(core_axis_name="c", subcore_axis_name="s",
                                    num_cores=2, num_subcores=16),
        scratch_types=[
            pltpu.VMEM((RCH,), jnp.int32),
            pltpu.VMEM((RCH,), jnp.int32),
            pltpu.VMEM((RCH, H), jnp.float32),
            pltpu.SemaphoreType.DMA,
            pltpu.SemaphoreType.DMA,
        ],
    )
    return run(x, d1, d2)


def _glu_body(tid_ref, valid_ref, xs_ref, gw_ref, uw_ref, dw_ref, ys_ref,
              gwb_ref, uwb_ref, dwb_ref, prev_ref):
    j = pl.program_id(0)

    @pl.when(valid_ref[j] > 0)
    def _():
        @pl.when((j == 0) | (tid_ref[j] != prev_ref[0]))
        def _():
            gwb_ref[...] = gw_ref[0].astype(jnp.bfloat16)
            uwb_ref[...] = uw_ref[0].astype(jnp.bfloat16)
            dwb_ref[...] = dw_ref[0].astype(jnp.bfloat16)

        prev_ref[0] = tid_ref[j]
        xb = xs_ref[...].astype(jnp.bfloat16)
        g = lax.dot_general(xb, gwb_ref[...], (((1,), (1,)), ((), ())),
                            preferred_element_type=jnp.float32)
        u = lax.dot_general(xb, uwb_ref[...], (((1,), (1,)), ((), ())),
                            preferred_element_type=jnp.float32)
        hh = ((g * jax.nn.sigmoid(g)) * u).astype(jnp.bfloat16)
        ys_ref[...] = lax.dot_general(hh, dwb_ref[...], (((1,), (1,)), ((), ())),
                                      preferred_element_type=jnp.float32)


def _grouped_glu(tile_eid, tile_valid, xs, gate_proj_w, up_proj_w,
                 down_proj_w):
    grid_spec = pltpu.PrefetchScalarGridSpec(
        num_scalar_prefetch=2,
        grid=(NT,),
        in_specs=[
            pl.BlockSpec((G, H), lambda j, tid, val: (j, 0)),
            pl.BlockSpec((1, I, H), lambda j, tid, val: (tid[j], 0, 0)),
            pl.BlockSpec((1, I, H), lambda j, tid, val: (tid[j], 0, 0)),
            pl.BlockSpec((1, H, I), lambda j, tid, val: (tid[j], 0, 0)),
        ],
        out_specs=pl.BlockSpec((G, H), lambda j, tid, val: (j, 0)),
        scratch_shapes=[
            pltpu.VMEM((I, H), jnp.bfloat16),
            pltpu.VMEM((I, H), jnp.bfloat16),
            pltpu.VMEM((H, I), jnp.bfloat16),
            pltpu.SMEM((1,), jnp.int32),
        ],
    )
    return pl.pallas_call(
        _glu_body,
        grid_spec=grid_spec,
        out_shape=jax.ShapeDtypeStruct((C, H), jnp.float32),
        compiler_params=pltpu.CompilerParams(
            dimension_semantics=("arbitrary",)),
    )(tile_eid, tile_valid, xs, gate_proj_w, up_proj_w, down_proj_w)


def _combine_body(ys_hbm, d1_hbm, d2_hbm, wbr_hbm,
                  out_hbm, da_v, db_v, wb_v, rowsa_v, rowsb_v,
                  out_v, sem, semb):
    wid = lax.axis_index("s") * 2 + lax.axis_index("c")
    base = wid * TPW
    for h in range(TPW // RCH):
        sub = base + h * RCH
        pltpu.sync_copy(d1_hbm.at[pl.ds(sub, RCH)], da_v)
        pltpu.sync_copy(d2_hbm.at[pl.ds(sub, RCH)], db_v)
        pltpu.sync_copy(wbr_hbm.at[pl.ds(sub, RCH)], wb_v)
        cpa = pltpu.async_copy(ys_hbm.at[da_v], rowsa_v, sem)
        cpb = pltpu.async_copy(ys_hbm.at[db_v], rowsb_v, semb)
        cpa.wait()
        cpb.wait()

        def jbody(j, _):
            wa = wb_v[j, pl.ds(0, 16)]
            wb = wb_v[j, pl.ds(16, 16)]
            for c in range(H // 16):
                sl = pl.ds(c * 16, 16)
                out_v[j, sl] = rowsa_v[j, sl] * wa + rowsb_v[j, sl] * wb
            return 0

        lax.fori_loop(0, RCH, jbody, 0)
        pltpu.sync_copy(out_v, out_hbm.at[pl.ds(sub, RCH)])


def _combine(ys, d1, d2, wbr):
    run = pl.kernel(
        _combine_body,
        out_type=jax.ShapeDtypeStruct((T, H), jnp.float32),
        mesh=plsc.VectorSubcoreMesh(core_axis_name="c", subcore_axis_name="s",
                                    num_cores=2, num_subcores=16),
        scratch_types=[
            pltpu.VMEM((RCH,), jnp.int32),
            pltpu.VMEM((RCH,), jnp.int32),
            pltpu.VMEM((RCH, 32), jnp.float32),
            pltpu.VMEM((RCH, H), jnp.float32),
            pltpu.VMEM((RCH, H), jnp.float32),
            pltpu.VMEM((RCH, H), jnp.float32),
            pltpu.SemaphoreType.DMA,
            pltpu.SemaphoreType.DMA,
        ],
    )
    return run(ys, d1, d2, wbr)


def _round_i32(x):
    return (x + 0.5).astype(jnp.int32)


def kernel(hidden_states, gate_w, gate_proj_w, up_proj_w, down_proj_w):
    b, s, hd = hidden_states.shape
    x = hidden_states.reshape(-1, hd)

    dests, wbr, counts_w = _router(x, gate_w)
    d1 = _round_i32(dests[:, 0])
    d2 = _round_i32(dests[:, 1])
    counts = _round_i32(counts_w[0])

    padded = ((counts + G - 1) // G) * G
    bounds = jnp.cumsum(padded)
    tiles_active = bounds[-1] // G
    tj = jnp.arange(NT, dtype=jnp.int32)
    eid_raw = jnp.sum((tj[:, None] * G >= bounds[None, :]).astype(jnp.int32),
                      axis=1)
    tile_valid = (tj < tiles_active).astype(jnp.int32)
    last_e = eid_raw[jnp.maximum(tiles_active - 1, 0)]
    tile_eid = jnp.where(tile_valid > 0, eid_raw, last_e)

    xs = _dispatch(x, d1, d2)
    ys = _grouped_glu(tile_eid, tile_valid, xs, gate_proj_w, up_proj_w,
                      down_proj_w)
    out = _combine(ys, d1, d2, wbr)
    return out.reshape(b, s, hd)


# speedup vs baseline: 1.3468x; 1.1159x over previous
import functools

import jax
import jax.numpy as jnp
from jax import lax
from jax.experimental import pallas as pl
from jax.experimental.pallas import tpu as pltpu
from jax.experimental.pallas import tpu_sc as plsc

E = 8
T = 2048
H = 1024
I = 768
G = 256
NT = (2 * T + E * (G - 1) + G - 1) // G
C = NT * G
TB = 512
NW = 32
TPW = T // NW
RCH = 32


def _router_body(x_ref, gate_ref, dests_ref, wbr_ref, counts_ref, carry_ref,
                 ls_ref):
    i = pl.program_id(0)

    @pl.when(i == 0)
    def _():
        carry_ref[...] = jnp.zeros_like(carry_ref)

    x = x_ref[...]
    logits = lax.dot_general(x, gate_ref[...], (((1,), (1,)), ((), ())),
                             preferred_element_type=jnp.float32)
    eio = lax.broadcasted_iota(jnp.int32, logits.shape, 1)
    m1 = jnp.max(logits, axis=1, keepdims=True)
    i1 = jnp.min(jnp.where(logits == m1, eio, E), axis=1, keepdims=True)
    l2 = jnp.where(eio == i1, -jnp.inf, logits)
    m2 = jnp.max(l2, axis=1, keepdims=True)
    i2 = jnp.min(jnp.where(l2 == m2, eio, E), axis=1, keepdims=True)
    w1 = 1.0 / (1.0 + jnp.exp(m2 - m1))
    w2 = 1.0 - w1

    o1 = (eio == i1).astype(jnp.float32)
    o2 = (eio == i2).astype(jnp.float32)
    osum = o1 + o2
    @pl.when(i == 0)
    def _():
        rr = lax.broadcasted_iota(jnp.int32, (TB, TB), 0)
        cc = lax.broadcasted_iota(jnp.int32, (TB, TB), 1)
        ls_ref[...] = (rr > cc).astype(jnp.float32)

    ls = ls_ref[...]
    carry = carry_ref[...]
    cum = lax.dot_general(ls, osum, (((1,), (0,)), ((), ())),
                          preferred_element_type=jnp.float32) + carry
    cb1 = jnp.sum(o1 * cum, axis=1, keepdims=True)
    cb2 = jnp.sum(o2 * cum, axis=1, keepdims=True)
    carry_ref[...] = carry + jnp.sum(osum, axis=0, keepdims=True)

    meta = jnp.concatenate(
        [cb1, cb2, i1.astype(jnp.float32), i2.astype(jnp.float32), w1, w2,
         jnp.zeros((TB, 2), jnp.float32)], axis=1)
    dests_ref[pl.ds(i * TB, TB), :] = meta

    @pl.when(i == T // TB - 1)
    def _():
        cnt = carry_ref[...]
        counts_ref[...] = cnt
        padded = jnp.floor((cnt + (G - 1)) / G) * G
        rr8 = lax.broadcasted_iota(jnp.int32, (E, E), 0)
        cc8 = lax.broadcasted_iota(jnp.int32, (E, E), 1)
        uincl = (rr8 <= cc8).astype(jnp.float32)
        bounds = lax.dot_general(padded, uincl, (((1,), (0,)), ((), ())),
                                 preferred_element_type=jnp.float32)
        offs = bounds - padded
        allm = dests_ref[...]
        eiof = lax.broadcasted_iota(jnp.int32, (T, E), 1).astype(jnp.float32)
        oo1 = (eiof == allm[:, 2:3]).astype(jnp.float32)
        oo2 = (eiof == allm[:, 3:4]).astype(jnp.float32)
        d1 = allm[:, 0:1] + jnp.sum(oo1 * offs, axis=1, keepdims=True)
        d2 = allm[:, 1:2] + jnp.sum(oo2 * offs, axis=1, keepdims=True)
        dests_ref[...] = jnp.concatenate(
            [d1, d2, allm[:, 2:]], axis=1)
        wbr_ref[...] = jnp.concatenate(
            [jnp.broadcast_to(allm[:, 4:5], (T, 16)),
             jnp.broadcast_to(allm[:, 5:6], (T, 16))], axis=1)


def _router(x, gate_w):
    return pl.pallas_call(
        _router_body,
        grid=(T // TB,),
        in_specs=[
            pl.BlockSpec((TB, H), lambda i: (i, 0)),
            pl.BlockSpec((E, H), lambda i: (0, 0)),
        ],
        out_specs=[
            pl.BlockSpec((T, 8), lambda i: (0, 0)),
            pl.BlockSpec((T, 32), lambda i: (0, 0)),
            pl.BlockSpec((1, E), lambda i: (0, 0)),
        ],
        out_shape=[
            jax.ShapeDtypeStruct((T, 8), jnp.float32),
            jax.ShapeDtypeStruct((T, 32), jnp.float32),
            jax.ShapeDtypeStruct((1, E), jnp.float32),
        ],
        scratch_shapes=[pltpu.VMEM((1, E), jnp.float32),
                        pltpu.VMEM((TB, TB), jnp.float32)],
        compiler_params=pltpu.CompilerParams(
            dimension_semantics=("arbitrary",)),
    )(x, gate_w)


def _dispatch_body(x_hbm, d1_hbm, d2_hbm, xs_hbm, da_v, db_v, rows_v, sem,
                   semb):
    wid = lax.axis_index("s") * 2 + lax.axis_index("c")
    base = wid * TPW
    for h in range(TPW // RCH):
        sub = base + h * RCH
        pltpu.sync_copy(x_hbm.at[pl.ds(sub, RCH)], rows_v)
        pltpu.sync_copy(d1_hbm.at[pl.ds(sub, RCH)], da_v)
        pltpu.sync_copy(d2_hbm.at[pl.ds(sub, RCH)], db_v)
        cpa = pltpu.async_copy(rows_v, xs_hbm.at[da_v], sem)
        cpb = pltpu.async_copy(rows_v, xs_hbm.at[db_v], semb)
        cpa.wait()
        cpb.wait()


def _dispatch(x, d1, d2):
    run = pl.kernel(
        _dispatch_body,
        out_type=jax.ShapeDtypeStruct((C, H), jnp.float32),
        mesh=plsc.VectorSubcoreMesh(core_axis_name="c", subcore_axis_name="s",
                                    num_cores=2, num_subcores=16),
        scratch_types=[
            pltpu.VMEM((RCH,), jnp.int32),
            pltpu.VMEM((RCH,), jnp.int32),
            pltpu.VMEM((RCH, H), jnp.float32),
            pltpu.SemaphoreType.DMA,
            pltpu.SemaphoreType.DMA,
        ],
    )
    return run(x, d1, d2)


def _glu_body(tid_ref, valid_ref, head_ref, nxt_ref, hasn_ref,
              xs_ref, gw_hbm, uw_hbm, dw_hbm, ys_ref,
              lg_ref, lu_ref, ld_ref, gwb_ref, uwb_ref, dwb_ref, par_ref,
              sg, su, sd):
    j = pl.program_id(0)
    tid = tid_ref[j]

    @pl.when(j == 0)
    def _():
        par_ref[0] = 0
        pltpu.make_async_copy(gw_hbm.at[tid], lg_ref.at[0], sg).start()
        pltpu.make_async_copy(uw_hbm.at[tid], lu_ref.at[0], su).start()
        pltpu.make_async_copy(dw_hbm.at[tid], ld_ref.at[0], sd).start()

    @pl.when((valid_ref[j] > 0) & (head_ref[j] > 0))
    def _():
        par = par_ref[0]
        pltpu.make_async_copy(gw_hbm.at[tid], lg_ref.at[par], sg).wait()
        pltpu.make_async_copy(uw_hbm.at[tid], lu_ref.at[par], su).wait()
        pltpu.make_async_copy(dw_hbm.at[tid], ld_ref.at[par], sd).wait()
        gwb_ref[...] = lg_ref[par].astype(jnp.bfloat16)
        uwb_ref[...] = lu_ref[par].astype(jnp.bfloat16)
        dwb_ref[...] = ld_ref[par].astype(jnp.bfloat16)

        @pl.when(hasn_ref[j] > 0)
        def _():
            nx = nxt_ref[j]
            pltpu.make_async_copy(gw_hbm.at[nx], lg_ref.at[1 - par], sg).start()
            pltpu.make_async_copy(uw_hbm.at[nx], lu_ref.at[1 - par], su).start()
            pltpu.make_async_copy(dw_hbm.at[nx], ld_ref.at[1 - par], sd).start()

        par_ref[0] = 1 - par

    @pl.when(valid_ref[j] > 0)
    def _():
        xb = xs_ref[...].astype(jnp.bfloat16)
        g = lax.dot_general(xb, gwb_ref[...], (((1,), (1,)), ((), ())),
                            preferred_element_type=jnp.float32)
        u = lax.dot_general(xb, uwb_ref[...], (((1,), (1,)), ((), ())),
                            preferred_element_type=jnp.float32)
        hh = ((g * jax.nn.sigmoid(g)) * u).astype(jnp.bfloat16)
        ys_ref[...] = lax.dot_general(hh, dwb_ref[...], (((1,), (1,)), ((), ())),
                                      preferred_element_type=jnp.float32)


def _grouped_glu(tile_eid, tile_valid, run_head, nxt_eid, has_next, xs,
                 gate_proj_w, up_proj_w, down_proj_w):
    grid_spec = pltpu.PrefetchScalarGridSpec(
        num_scalar_prefetch=5,
        grid=(NT,),
        in_specs=[
            pl.BlockSpec((G, H), lambda j, *_: (j, 0)),
            pl.BlockSpec(memory_space=pl.ANY),
            pl.BlockSpec(memory_space=pl.ANY),
            pl.BlockSpec(memory_space=pl.ANY),
        ],
        out_specs=pl.BlockSpec((G, H), lambda j, *_: (j, 0)),
        scratch_shapes=[
            pltpu.VMEM((2, I, H), jnp.float32),
            pltpu.VMEM((2, I, H), jnp.float32),
            pltpu.VMEM((2, H, I), jnp.float32),
            pltpu.VMEM((I, H), jnp.bfloat16),
            pltpu.VMEM((I, H), jnp.bfloat16),
            pltpu.VMEM((H, I), jnp.bfloat16),
            pltpu.SMEM((1,), jnp.int32),
            pltpu.SemaphoreType.DMA,
            pltpu.SemaphoreType.DMA,
            pltpu.SemaphoreType.DMA,
        ],
    )
    return pl.pallas_call(
        _glu_body,
        grid_spec=grid_spec,
        out_shape=jax.ShapeDtypeStruct((C, H), jnp.float32),
        compiler_params=pltpu.CompilerParams(
            dimension_semantics=("arbitrary",)),
    )(tile_eid, tile_valid, run_head, nxt_eid, has_next, xs,
      gate_proj_w, up_proj_w, down_proj_w)


def _combine_body(ys_hbm, d1_hbm, d2_hbm, wbr_hbm,
                  out_hbm, da_v, db_v, wb_v, rowsa_v, rowsb_v,
                  out_v, sem, semb):
    wid = lax.axis_index("s") * 2 + lax.axis_index("c")
    base = wid * TPW
    for h in range(TPW // RCH):
        sub = base + h * RCH
        pltpu.sync_copy(d1_hbm.at[pl.ds(sub, RCH)], da_v)
        pltpu.sync_copy(d2_hbm.at[pl.ds(sub, RCH)], db_v)
        pltpu.sync_copy(wbr_hbm.at[pl.ds(sub, RCH)], wb_v)
        cpa = pltpu.async_copy(ys_hbm.at[da_v], rowsa_v, sem)
        cpb = pltpu.async_copy(ys_hbm.at[db_v], rowsb_v, semb)
        cpa.wait()
        cpb.wait()

        def jbody(j, _):
            wa = wb_v[j, pl.ds(0, 16)]
            wb = wb_v[j, pl.ds(16, 16)]
            for c in range(H // 16):
                sl = pl.ds(c * 16, 16)
                out_v[j, sl] = rowsa_v[j, sl] * wa + rowsb_v[j, sl] * wb
            return 0

        lax.fori_loop(0, RCH, jbody, 0)
        pltpu.sync_copy(out_v, out_hbm.at[pl.ds(sub, RCH)])


def _combine(ys, d1, d2, wbr):
    run = pl.kernel(
        _combine_body,
        out_type=jax.ShapeDtypeStruct((T, H), jnp.float32),
        mesh=plsc.VectorSubcoreMesh(core_axis_name="c", subcore_axis_name="s",
                                    num_cores=2, num_subcores=16),
        scratch_types=[
            pltpu.VMEM((RCH,), jnp.int32),
            pltpu.VMEM((RCH,), jnp.int32),
            pltpu.VMEM((RCH, 32), jnp.float32),
            pltpu.VMEM((RCH, H), jnp.float32),
            pltpu.VMEM((RCH, H), jnp.float32),
            pltpu.VMEM((RCH, H), jnp.float32),
            pltpu.SemaphoreType.DMA,
            pltpu.SemaphoreType.DMA,
        ],
    )
    return run(ys, d1, d2, wbr)


def _round_i32(x):
    return (x + 0.5).astype(jnp.int32)


def kernel(hidden_states, gate_w, gate_proj_w, up_proj_w, down_proj_w):
    b, s, hd = hidden_states.shape
    x = hidden_states.reshape(-1, hd)

    dests, wbr, counts_w = _router(x, gate_w)
    d1 = _round_i32(dests[:, 0])
    d2 = _round_i32(dests[:, 1])
    counts = _round_i32(counts_w[0])

    padded = ((counts + G - 1) // G) * G
    bounds = jnp.cumsum(padded)
    tiles_active = bounds[-1] // G
    tj = jnp.arange(NT, dtype=jnp.int32)
    eid_raw = jnp.sum((tj[:, None] * G >= bounds[None, :]).astype(jnp.int32),
                      axis=1)
    tile_valid = (tj < tiles_active).astype(jnp.int32)
    last_e = eid_raw[jnp.maximum(tiles_active - 1, 0)]
    tile_eid = jnp.where(tile_valid > 0, eid_raw, last_e)
    run_head = jnp.concatenate(
        [jnp.ones((1,), jnp.int32),
         (tile_eid[1:] != tile_eid[:-1]).astype(jnp.int32)])
    kk = jnp.where(run_head > 0, tj, 2 * NT)
    sufmin = lax.cummin(kk[::-1])[::-1]
    nxt_pos = jnp.concatenate([sufmin[1:], jnp.full((1,), 2 * NT, jnp.int32)])
    has_next = (nxt_pos < NT).astype(jnp.int32)
    nxt_eid = tile_eid[jnp.clip(nxt_pos, 0, NT - 1)]

    xs = _dispatch(x, d1, d2)
    ys = _grouped_glu(tile_eid, tile_valid, run_head, nxt_eid, has_next, xs,
                      gate_proj_w, up_proj_w, down_proj_w)
    out = _combine(ys, d1, d2, wbr)
    return out.reshape(b, s, hd)
